# baseline (speedup 1.0000x reference)
import jax
import jax.numpy as jnp
from jax import lax
from jax.experimental import pallas as pl
from jax.experimental.pallas import tpu as pltpu
from jax.experimental.pallas import tpu_sc as plsc

N_NODES = 10000
D_FEAT = 128
E = 320000
E_ALL = 2 * E
NC = 2
NS = 16
L = 16
FPT = D_FEAT // NS
WPT = FPT // 2
STRIDE = WPT + 1
CHUNK = 3200
N_CHUNKS = E // CHUNK
N_CHUNKS_ALL = NC * N_CHUNKS
_EPS = 1e-15


def _dot_body(z_hbm, edges_hbm, part_hbm, zt, src_v, dst_v, part_v):
    c = lax.axis_index("c")
    s = lax.axis_index("s")
    pltpu.sync_copy(z_hbm.at[s], zt)
    base = c * N_CHUNKS

    def chunk_body(k, carry):
        off = (base + k) * CHUNK
        pltpu.sync_copy(edges_hbm.at[0, pl.ds(off, CHUNK)], src_v)
        pltpu.sync_copy(edges_hbm.at[1, pl.ds(off, CHUNK)], dst_v)

        @plsc.parallel_loop(0, CHUNK // L, unroll=4)
        def grp(g):
            si = src_v[pl.ds(g * L, L)] * STRIDE
            di = dst_v[pl.ds(g * L, L)] * STRIDE
            acc = None
            for w in range(WPT):
                sw = plsc.bitcast(plsc.load_gather(zt, [si + w]),
                                  jnp.bfloat16)
                dw = plsc.bitcast(plsc.load_gather(zt, [di + w]),
                                  jnp.bfloat16)
                prod = sw * dw
                acc = prod if acc is None else acc + prod
            lo, hi = plsc.unpack(acc, format=plsc.PackFormat.INTERLEAVED)
            part_v[pl.ds(g * L, L)] = lo + hi

        pltpu.sync_copy(part_v, part_hbm.at[base + k, s])
        return carry

    lax.fori_loop(0, N_CHUNKS, chunk_body, 0)


SUB = 8
CPS = CHUNK // SUB

_dot_kernel = pl.kernel(
    _dot_body,
    out_type=jax.ShapeDtypeStruct((N_CHUNKS_ALL, NS, CHUNK), jnp.float32),
    mesh=plsc.VectorSubcoreMesh(core_axis_name="c", subcore_axis_name="s"),
    scratch_types=[
        pltpu.VMEM((N_NODES * STRIDE,), jnp.int32),
        pltpu.VMEM((CHUNK,), jnp.int32),
        pltpu.VMEM((CHUNK,), jnp.int32),
        pltpu.VMEM((CHUNK,), jnp.float32),
    ],
    compiler_params=pltpu.CompilerParams(needs_layout_passes=False),
)

N_POS_BLOCKS = N_CHUNKS


def _loss_body(p_ref, o_ref):
    i = pl.program_id(0)
    v = jnp.sum(p_ref[0], axis=0)
    p = 1.0 / (1.0 + jnp.exp(-v))

    @pl.when(i == 0)
    def _init():
        o_ref[...] = jnp.zeros_like(o_ref)

    @pl.when(i < N_POS_BLOCKS)
    def _pos():
        o_ref[...] += jnp.sum(-jnp.log(p + _EPS)) * (1.0 / E)

    @pl.when(i >= N_POS_BLOCKS)
    def _neg():
        o_ref[...] += jnp.sum(-jnp.log(1.0 - p + _EPS)) * (1.0 / E)


_loss_kernel = pl.pallas_call(
    _loss_body,
    grid=(N_CHUNKS_ALL,),
    in_specs=[pl.BlockSpec((1, NS, SUB, CPS), lambda i: (i, 0, 0, 0))],
    out_specs=pl.BlockSpec((1, 1), lambda i: (0, 0)),
    out_shape=jax.ShapeDtypeStruct((1, 1), jnp.float32),
)


def kernel(z, pos_edge_index, neg_edge_index):
    zb = z.astype(jnp.bfloat16).reshape(N_NODES, NS, WPT, 2)
    zw = lax.bitcast_convert_type(zb, jnp.int32)
    zw = jnp.pad(zw, ((0, 0), (0, 0), (0, STRIDE - WPT)))
    zt = zw.transpose(1, 0, 2).reshape(NS, N_NODES * STRIDE)
    edges = jnp.concatenate(
        [pos_edge_index, neg_edge_index], axis=1).astype(jnp.int32)
    parts = _dot_kernel(zt, edges)
    parts = parts.reshape(N_CHUNKS_ALL, NS, SUB, CPS)
    return _loss_kernel(parts)[0, 0]


# speedup vs baseline: 1.1034x; 1.1034x over previous
import jax
import jax.numpy as jnp
from jax import lax
from jax.experimental import pallas as pl
from jax.experimental.pallas import tpu as pltpu
from jax.experimental.pallas import tpu_sc as plsc

N_NODES = 10000
D_FEAT = 128
E = 320000
E_ALL = 2 * E
NC = 2
NS = 16
L = 16
FPT = D_FEAT // NS
WPT = FPT // 2
STRIDE = WPT + 1
CHUNK = 3200
N_CHUNKS = E // CHUNK
N_CHUNKS_ALL = NC * N_CHUNKS
SUB = 8
CPS = CHUNK // SUB
GPR = CPS // L
_EPS = 1e-15


def _dot_body(z_hbm, edges_hbm, part_hbm, zt, src_v, dst_v, part_v):
    c = lax.axis_index("c")
    s = lax.axis_index("s")
    pltpu.sync_copy(z_hbm.at[s], zt)
    base = c * N_CHUNKS

    def chunk_body(k, carry):
        off = (base + k) * CHUNK
        pltpu.sync_copy(edges_hbm.at[0, pl.ds(off, CHUNK)], src_v)
        pltpu.sync_copy(edges_hbm.at[1, pl.ds(off, CHUNK)], dst_v)

        @plsc.parallel_loop(0, CHUNK // L, unroll=4)
        def grp(g):
            si = src_v[pl.ds(g * L, L)] * STRIDE
            di = dst_v[pl.ds(g * L, L)] * STRIDE
            acc = None
            for w in range(WPT):
                sw = plsc.bitcast(plsc.load_gather(zt, [si + w]),
                                  jnp.bfloat16)
                dw = plsc.bitcast(plsc.load_gather(zt, [di + w]),
                                  jnp.bfloat16)
                prod = sw * dw
                acc = prod if acc is None else acc + prod
            lo, hi = plsc.unpack(acc, format=plsc.PackFormat.INTERLEAVED)
            r = g // GPR
            part_v[r, pl.ds((g - r * GPR) * L, L)] = lo + hi

        pltpu.sync_copy(part_v, part_hbm.at[base + k, s])
        return carry

    lax.fori_loop(0, N_CHUNKS, chunk_body, 0)


_dot_kernel = pl.kernel(
    _dot_body,
    out_type=jax.ShapeDtypeStruct((N_CHUNKS_ALL, NS, SUB, CPS), jnp.float32),
    mesh=plsc.VectorSubcoreMesh(core_axis_name="c", subcore_axis_name="s"),
    scratch_types=[
        pltpu.VMEM((N_NODES * STRIDE,), jnp.int32),
        pltpu.VMEM((CHUNK,), jnp.int32),
        pltpu.VMEM((CHUNK,), jnp.int32),
        pltpu.VMEM((SUB, CPS), jnp.float32),
    ],
    compiler_params=pltpu.CompilerParams(needs_layout_passes=False),
)

N_POS_BLOCKS = N_CHUNKS


def _loss_body(p_ref, o_ref):
    i = pl.program_id(0)
    v = jnp.sum(p_ref[0], axis=0)
    p = 1.0 / (1.0 + jnp.exp(-v))

    @pl.when(i == 0)
    def _init():
        o_ref[...] = jnp.zeros_like(o_ref)

    @pl.when(i < N_POS_BLOCKS)
    def _pos():
        o_ref[...] += jnp.sum(-jnp.log(p + _EPS)) * (1.0 / E)

    @pl.when(i >= N_POS_BLOCKS)
    def _neg():
        o_ref[...] += jnp.sum(-jnp.log(1.0 - p + _EPS)) * (1.0 / E)


_loss_kernel = pl.pallas_call(
    _loss_body,
    grid=(N_CHUNKS_ALL,),
    in_specs=[pl.BlockSpec((1, NS, SUB, CPS), lambda i: (i, 0, 0, 0))],
    out_specs=pl.BlockSpec((1, 1), lambda i: (0, 0)),
    out_shape=jax.ShapeDtypeStruct((1, 1), jnp.float32),
)


def kernel(z, pos_edge_index, neg_edge_index):
    zb = z.astype(jnp.bfloat16).reshape(N_NODES, NS, WPT, 2)
    zw = lax.bitcast_convert_type(zb, jnp.int32)
    zw = jnp.pad(zw, ((0, 0), (0, 0), (0, STRIDE - WPT)))
    zt = zw.transpose(1, 0, 2).reshape(NS, N_NODES * STRIDE)
    edges = jnp.concatenate(
        [pos_edge_index, neg_edge_index], axis=1).astype(jnp.int32)
    parts = _dot_kernel(zt, edges)
    return _loss_kernel(parts)[0, 0]


# speedup vs baseline: 1.1121x; 1.0078x over previous
import jax
import jax.numpy as jnp
from jax import lax
from jax.experimental import pallas as pl
from jax.experimental.pallas import tpu as pltpu
from jax.experimental.pallas import tpu_sc as plsc

N_NODES = 10000
D_FEAT = 128
E = 320000
E_ALL = 2 * E
NC = 2
NS = 16
L = 16
FPT = D_FEAT // NS
WPT = FPT // 2
STRIDE = WPT + 1
CHUNK = 3200
N_CHUNKS = E // CHUNK
N_CHUNKS_ALL = NC * N_CHUNKS
_EPS = 1e-15


def _dot_body(z_hbm, edges_hbm, part_hbm, zt, src_v, dst_v, part_v):
    c = lax.axis_index("c")
    s = lax.axis_index("s")
    pltpu.sync_copy(z_hbm.at[s], zt)
    base = c * N_CHUNKS

    def chunk_body(k, carry):
        off = (base + k) * CHUNK
        pltpu.sync_copy(edges_hbm.at[0, pl.ds(off, CHUNK)], src_v)
        pltpu.sync_copy(edges_hbm.at[1, pl.ds(off, CHUNK)], dst_v)

        @plsc.parallel_loop(0, CHUNK // L, unroll=4)
        def grp(g):
            si = src_v[pl.ds(g * L, L)] * STRIDE
            di = dst_v[pl.ds(g * L, L)] * STRIDE
            acc = None
            for w in range(WPT):
                sw = plsc.bitcast(plsc.load_gather(zt, [si + w]),
                                  jnp.bfloat16)
                dw = plsc.bitcast(plsc.load_gather(zt, [di + w]),
                                  jnp.bfloat16)
                prod = sw * dw
                acc = prod if acc is None else acc + prod
            lo, hi = plsc.unpack(acc, format=plsc.PackFormat.INTERLEAVED)
            part_v[pl.ds(g * L, L)] = lo + hi

        pltpu.sync_copy(part_v, part_hbm.at[base + k, s])
        return carry

    lax.fori_loop(0, N_CHUNKS, chunk_body, 0)


_dot_kernel = pl.kernel(
    _dot_body,
    out_type=jax.ShapeDtypeStruct((N_CHUNKS_ALL, NS, CHUNK), jnp.float32),
    mesh=plsc.VectorSubcoreMesh(core_axis_name="c", subcore_axis_name="s"),
    scratch_types=[
        pltpu.VMEM((N_NODES * STRIDE,), jnp.int32),
        pltpu.VMEM((CHUNK,), jnp.int32),
        pltpu.VMEM((CHUNK,), jnp.int32),
        pltpu.VMEM((CHUNK,), jnp.float32),
    ],
    compiler_params=pltpu.CompilerParams(needs_layout_passes=False),
)

N_POS_BLOCKS = N_CHUNKS


def _loss_body(p_ref, o_ref):
    i = pl.program_id(0)
    v = jnp.sum(p_ref[0], axis=0, keepdims=True)
    p = 1.0 / (1.0 + jnp.exp(-v))
    t = jnp.where(i < N_POS_BLOCKS,
                  -jnp.log(p + _EPS),
                  -jnp.log(1.0 - p + _EPS))
    val = jnp.sum(t) * (1.0 / E)

    @pl.when(i == 0)
    def _init():
        o_ref[...] = jnp.zeros_like(o_ref)

    o_ref[...] += val


_loss_kernel = pl.pallas_call(
    _loss_body,
    grid=(N_CHUNKS_ALL,),
    in_specs=[pl.BlockSpec((1, NS, CHUNK), lambda i: (i, 0, 0))],
    out_specs=pl.BlockSpec((1, 1), lambda i: (0, 0)),
    out_shape=jax.ShapeDtypeStruct((1, 1), jnp.float32),
)


def kernel(z, pos_edge_index, neg_edge_index):
    zb = z.astype(jnp.bfloat16).reshape(N_NODES, NS, WPT, 2)
    zw = lax.bitcast_convert_type(zb, jnp.int32)
    zw = jnp.pad(zw, ((0, 0), (0, 0), (0, STRIDE - WPT)))
    zt = zw.transpose(1, 0, 2).reshape(NS, N_NODES * STRIDE)
    edges = jnp.concatenate(
        [pos_edge_index, neg_edge_index], axis=1).astype(jnp.int32)
    parts = _dot_kernel(zt, edges)
    return _loss_kernel(parts)[0, 0]
